# baseline (speedup 1.0000x reference)
import jax
import jax.numpy as jnp
from jax import lax
from jax.experimental import pallas as pl
from jax.experimental.pallas import tpu as pltpu
from jax.experimental.pallas import tpu_sc as plsc

NC, NS, L = 2, 16, 16
NW = NC * NS
NQ = 16384
NX = 4096
NXP = NX + 8 * L
QPW = NQ // NW
NG = QPW // L
CHUNK = 128
NCHUNK = QPW // CHUNK
D = 128
INF = float("inf")
SENT = 1e18

_PERM_DN = lax.GatherDimensionNumbers(
    offset_dims=(), collapsed_slice_dims=(0,), start_index_map=(0,))


def _perm(v, idx):
    return lax.gather(v, idx[:, None], _PERM_DN, (1,),
                      mode=lax.GatherScatterMode.PROMISE_IN_BOUNDS)


def _ext(v, i):
    return lax.squeeze(lax.slice(v, (i,), (i + 1,)), (0,))


def _top3_update(dm, jv, d1, d2, d3, j1, j2, j3):
    c1 = dm < d1
    c2 = dm < d2
    c3 = dm < d3
    d3n = jnp.where(c3, jnp.where(c2, d2, dm), d3)
    j3n = jnp.where(c3, jnp.where(c2, j2, jv), j3)
    d2n = jnp.where(c2, jnp.where(c1, d1, dm), d2)
    j2n = jnp.where(c2, jnp.where(c1, j1, jv), j2)
    d1n = jnp.where(c1, dm, d1)
    j1n = jnp.where(c1, jv, j1)
    return d1n, d2n, d3n, j1n, j2n, j3n


def _merge_pop(a, b):
    (ad, aj), (bd, bj) = a, b
    c = bd[0] < ad[0]
    od = jnp.where(c, bd[0], ad[0])
    oj = jnp.where(c, bj[0], aj[0])
    an = ([jnp.where(c, x, y) for x, y in zip(ad[:-1], ad[1:])],
          [jnp.where(c, x, y) for x, y in zip(aj[:-1], aj[1:])])
    bn = ([jnp.where(c, y, x) for x, y in zip(bd[:-1], bd[1:])],
          [jnp.where(c, y, x) for x, y in zip(bj[:-1], bj[1:])])
    return od, oj, an, bn


def _sc_knn_body(posx_h, posy_h, posz_h, qx_h, qy_h, qz_h, qb_h,
                 sp_h, ep_h, sh_h, glo_h, ghi_h, x_h, out_h,
                 posx, posy, posz, qx, qy, qz, qb, sp_r, ep_r, sh_r,
                 glo, ghi, wn1, wn2, wn3, i1, i2, i3,
                 rows1, rows2, rows3, obuf, sem):
    wid = lax.axis_index("s") * NC + lax.axis_index("c")
    base = wid * QPW

    pltpu.sync_copy(posx_h, posx)
    pltpu.sync_copy(posy_h, posy)
    pltpu.sync_copy(posz_h, posz)
    pltpu.sync_copy(qx_h.at[pl.ds(base, QPW)], qx)
    pltpu.sync_copy(qy_h.at[pl.ds(base, QPW)], qy)
    pltpu.sync_copy(qz_h.at[pl.ds(base, QPW)], qz)
    pltpu.sync_copy(qb_h.at[pl.ds(base, QPW)], qb)
    pltpu.sync_copy(sp_h, sp_r)
    pltpu.sync_copy(ep_h, ep_r)
    pltpu.sync_copy(sh_h, sh_r)
    pltpu.sync_copy(glo_h.at[pl.ds(wid * NG, NG)], glo.at[pl.ds(0, NG)])
    pltpu.sync_copy(ghi_h.at[pl.ds(wid * NG, NG)], ghi.at[pl.ds(0, NG)])

    spv = sp_r[pl.ds(0, L)]
    epv = ep_r[pl.ds(0, L)]
    shv = sh_r[pl.ds(0, L)]

    finf = jnp.full((L,), INF, jnp.float32)
    zi = jnp.zeros((L,), jnp.int32)

    @pl.loop(0, NG)
    def _group(g):
        gb = g * L
        qxg = qx[pl.ds(gb, L)]
        qyg = qy[pl.ds(gb, L)]
        qzg = qz[pl.ds(gb, L)]
        bg = qb[pl.ds(gb, L)]
        lob = _ext(glo[pl.ds(g, L)], 0)
        hib = _ext(ghi[pl.ds(g, L)], 0)
        uniform = _ext(bg, 0) == _ext(bg, L - 1)

        def scan(masked):
            if masked:
                los = _perm(spv, bg)
                his = _perm(epv, bg)

            init = (finf, finf, finf, zi, zi, zi,
                    finf, finf, finf, zi, zi, zi)

            @pl.loop(lob, hib, init_carry=init)
            def _blk(jb, carry):
                (d1a, d2a, d3a, j1a, j2a, j3a,
                 d1b, d2b, d3b, j1b, j2b, j3b) = carry
                jb16 = jb * L
                px16 = posx[pl.ds(jb16, L)]
                py16 = posy[pl.ds(jb16, L)]
                pz16 = posz[pl.ds(jb16, L)]
                jvb = jnp.full((L,), jb16, jnp.int32)
                dms = []
                for t in range(L):
                    tv = jnp.full((L,), t, jnp.int32)
                    px = _perm(px16, tv)
                    py = _perm(py16, tv)
                    pz = _perm(pz16, tv)
                    jv = jvb + t
                    dx = qxg - px
                    dy = qyg - py
                    dz = qzg - pz
                    d = dx * dx + dy * dy + dz * dz
                    if masked:
                        d = jnp.where((jv >= los) & (jv < his), d, INF)
                    dms.append((d, jv))
                for t in range(0, L, 2):
                    d1a, d2a, d3a, j1a, j2a, j3a = _top3_update(
                        dms[t][0], dms[t][1], d1a, d2a, d3a, j1a, j2a, j3a)
                    d1b, d2b, d3b, j1b, j2b, j3b = _top3_update(
                        dms[t + 1][0], dms[t + 1][1],
                        d1b, d2b, d3b, j1b, j2b, j3b)
                return (d1a, d2a, d3a, j1a, j2a, j3a,
                        d1b, d2b, d3b, j1b, j2b, j3b)

            (d1a, d2a, d3a, j1a, j2a, j3a,
             d1b, d2b, d3b, j1b, j2b, j3b) = _blk

            a = ([d1a, d2a, d3a], [j1a, j2a, j3a])
            b = ([d1b, d2b, d3b], [j1b, j2b, j3b])
            o1d, o1j, a2_, b2_ = _merge_pop(a, b)
            o2d, o2j, a3_, b3_ = _merge_pop(a2_, b2_)
            c = b3_[0][0] < a3_[0][0]
            o3d = jnp.where(c, b3_[0][0], a3_[0][0])
            o3j = jnp.where(c, b3_[1][0], a3_[1][0])

            jadj = _perm(shv, bg)
            o1jc = jnp.clip(o1j - jadj, 0, NX - 1)
            o2jc = jnp.clip(o2j - jadj, 0, NX - 1)
            o3jc = jnp.clip(o3j - jadj, 0, NX - 1)

            w1 = 1.0 / jnp.maximum(o1d, 1e-16)
            w2 = 1.0 / jnp.maximum(o2d, 1e-16)
            w3 = 1.0 / jnp.maximum(o3d, 1e-16)
            r = 1.0 / (w1 + w2 + w3)
            wn1[pl.ds(gb, L)] = w1 * r
            wn2[pl.ds(gb, L)] = w2 * r
            wn3[pl.ds(gb, L)] = w3 * r
            i1[pl.ds(gb, L)] = o1jc
            i2[pl.ds(gb, L)] = o2jc
            i3[pl.ds(gb, L)] = o3jc

        lax.cond(uniform, lambda: scan(False), lambda: scan(True))

    zsplat = jnp.zeros((L,), jnp.int32)
    for c in range(NCHUNK):
        cb = c * CHUNK
        cp1 = pltpu.async_copy(x_h.at[i1.at[pl.ds(cb, CHUNK)]], rows1, sem)
        cp2 = pltpu.async_copy(x_h.at[i2.at[pl.ds(cb, CHUNK)]], rows2, sem)
        cp3 = pltpu.async_copy(x_h.at[i3.at[pl.ds(cb, CHUNK)]], rows3, sem)
        cp1.wait()
        cp2.wait()
        cp3.wait()

        @pl.loop(0, CHUNK)
        def _combine(q):
            a1 = _perm(wn1[pl.ds(cb + q, L)], zsplat)
            a2 = _perm(wn2[pl.ds(cb + q, L)], zsplat)
            a3 = _perm(wn3[pl.ds(cb + q, L)], zsplat)
            for k in range(D // L):
                kk = pl.ds(k * L, L)
                obuf[q, kk] = (a1 * rows1[q, kk] + a2 * rows2[q, kk]
                               + a3 * rows3[q, kk])

        pltpu.sync_copy(obuf, out_h.at[pl.ds(base + cb, CHUNK)])


def _sc_knn(posx, posy, posz, qx, qy, qz, qb, sp, ep, sh, glo, ghi, x):
    mesh = plsc.VectorSubcoreMesh(core_axis_name="c", subcore_axis_name="s",
                                  num_cores=NC, num_subcores=NS)
    f = pl.kernel(
        _sc_knn_body,
        out_type=jax.ShapeDtypeStruct((NQ, D), jnp.float32),
        mesh=mesh,
        scratch_types=[
            pltpu.VMEM((NXP,), jnp.float32),
            pltpu.VMEM((NXP,), jnp.float32),
            pltpu.VMEM((NXP,), jnp.float32),
            pltpu.VMEM((QPW,), jnp.float32),
            pltpu.VMEM((QPW,), jnp.float32),
            pltpu.VMEM((QPW,), jnp.float32),
            pltpu.VMEM((QPW,), jnp.int32),
            pltpu.VMEM((L,), jnp.int32),
            pltpu.VMEM((L,), jnp.int32),
            pltpu.VMEM((L,), jnp.int32),
            pltpu.VMEM((NG + L,), jnp.int32),
            pltpu.VMEM((NG + L,), jnp.int32),
            pltpu.VMEM((QPW + L,), jnp.float32),
            pltpu.VMEM((QPW + L,), jnp.float32),
            pltpu.VMEM((QPW + L,), jnp.float32),
            pltpu.VMEM((QPW,), jnp.int32),
            pltpu.VMEM((QPW,), jnp.int32),
            pltpu.VMEM((QPW,), jnp.int32),
            pltpu.VMEM((CHUNK, D), jnp.float32),
            pltpu.VMEM((CHUNK, D), jnp.float32),
            pltpu.VMEM((CHUNK, D), jnp.float32),
            pltpu.VMEM((CHUNK, D), jnp.float32),
            pltpu.SemaphoreType.DMA,
        ],
    )
    return f(posx, posy, posz, qx, qy, qz, qb, sp, ep, sh, glo, ghi, x)


def _tc_mlp_body(xi_ref, xs_ref, pe_ref, w1a_ref, w1b_ref, b1_ref,
                 w2_ref, b2_ref, wp_ref, bp_ref, out_ref):
    i = pl.program_id(0)
    pe_row = pe_ref[pl.ds(i, 1), :]
    pr = jnp.dot(pe_row, wp_ref[...],
                 preferred_element_type=jnp.float32) + bp_ref[...]
    pr = jnp.maximum(pr, 0.0)
    h = jnp.dot(xi_ref[...], w1a_ref[...], preferred_element_type=jnp.float32)
    h = h + jnp.dot(xs_ref[...], w1b_ref[...],
                    preferred_element_type=jnp.float32)
    h = jnp.maximum(h + b1_ref[...], 0.0)
    h = jnp.dot(h, w2_ref[...], preferred_element_type=jnp.float32)
    out_ref[...] = (h + b2_ref[...]) * pr


def _tc_mlp(xi, xs, pe, w1a, w1b, b1, w2, b2, wp, bp):
    nb = 8
    rb = NQ // nb
    return pl.pallas_call(
        _tc_mlp_body,
        grid=(nb,),
        in_specs=[
            pl.BlockSpec((rb, 128), lambda i: (i, 0)),
            pl.BlockSpec((rb, 64), lambda i: (i, 0)),
            pl.BlockSpec((8, 512), lambda i: (0, 0)),
            pl.BlockSpec((128, 128), lambda i: (0, 0)),
            pl.BlockSpec((64, 128), lambda i: (0, 0)),
            pl.BlockSpec((1, 128), lambda i: (0, 0)),
            pl.BlockSpec((128, 128), lambda i: (0, 0)),
            pl.BlockSpec((1, 128), lambda i: (0, 0)),
            pl.BlockSpec((512, 128), lambda i: (0, 0)),
            pl.BlockSpec((1, 128), lambda i: (0, 0)),
        ],
        out_specs=pl.BlockSpec((rb, 128), lambda i: (i, 0)),
        out_shape=jax.ShapeDtypeStruct((NQ, 128), jnp.float32),
    )(xi, xs, pe, w1a, w1b, b1, w2, b2, wp, bp)


def kernel(par_embedding, x, pos, batch, x_skip, pos_skip, batch_skip,
           W1, b1, W2, b2, Wp, bp):
    batch = batch.astype(jnp.int32)
    qb = batch_skip.astype(jnp.int32)
    qx = pos_skip[:, 0] + 0.0
    qy = pos_skip[:, 1] + 0.0
    qz = pos_skip[:, 2] + 0.0
    ar = jnp.arange(8, dtype=jnp.int32)
    ss = jnp.searchsorted(batch, ar, side="left").astype(jnp.int32)
    se = jnp.searchsorted(batch, ar, side="right").astype(jnp.int32)
    cnt = se - ss
    cntp = ((cnt + L - 1) // L) * L
    off = jnp.concatenate([jnp.zeros((1,), jnp.int32),
                           jnp.cumsum(cntp)[:-1].astype(jnp.int32)])
    shift = off - ss
    dst = jnp.arange(NX, dtype=jnp.int32) + shift[batch]
    sent = jnp.full((NXP,), SENT, jnp.float32)
    posx = sent.at[dst].set(pos[:, 0])
    posy = sent.at[dst].set(pos[:, 1])
    posz = sent.at[dst].set(pos[:, 2])
    sp16 = jnp.pad(off, (0, L - 8))
    ep16 = jnp.pad(off + cnt, (0, L - 8))
    sh16 = jnp.pad(shift, (0, L - 8))
    bs = qb.reshape(NQ // L, L)
    glo = jnp.pad(off[bs[:, 0]] // L, (0, L))
    ghi = jnp.pad((off + cntp)[bs[:, L - 1]] // L, (0, L))

    xi = _sc_knn(posx, posy, posz, qx, qy, qz, qb,
                 sp16, ep16, sh16, glo, ghi, x)

    pe = par_embedding.reshape(8, 512)
    w1a = W1[:128]
    w1b = W1[128:]
    out = _tc_mlp(xi, x_skip, pe, w1a, w1b, b1.reshape(1, 128),
                  W2, b2.reshape(1, 128), Wp, bp.reshape(1, 128))
    return out, pos_skip, batch_skip


# speedup vs baseline: 1.0043x; 1.0043x over previous
import jax
import jax.numpy as jnp
from jax import lax
from jax.experimental import pallas as pl
from jax.experimental.pallas import tpu as pltpu
from jax.experimental.pallas import tpu_sc as plsc

NC, NS, L = 2, 16, 16
NW = NC * NS
NQ = 16384
NX = 4096
NXP = NX + 8 * L
QPW = NQ // NW
NG = QPW // L
CHUNK = 128
NCHUNK = QPW // CHUNK
D = 128
INF = float("inf")
SENT = 1e18

_PERM_DN = lax.GatherDimensionNumbers(
    offset_dims=(), collapsed_slice_dims=(0,), start_index_map=(0,))


def _perm(v, idx):
    return lax.gather(v, idx[:, None], _PERM_DN, (1,),
                      mode=lax.GatherScatterMode.PROMISE_IN_BOUNDS)


def _ext(v, i):
    return lax.squeeze(lax.slice(v, (i,), (i + 1,)), (0,))


def _top3_update(dm, jv, d1, d2, d3, j1, j2, j3):
    c1 = dm < d1
    c2 = dm < d2
    c3 = dm < d3
    d3n = jnp.where(c3, jnp.where(c2, d2, dm), d3)
    j3n = jnp.where(c3, jnp.where(c2, j2, jv), j3)
    d2n = jnp.where(c2, jnp.where(c1, d1, dm), d2)
    j2n = jnp.where(c2, jnp.where(c1, j1, jv), j2)
    d1n = jnp.where(c1, dm, d1)
    j1n = jnp.where(c1, jv, j1)
    return d1n, d2n, d3n, j1n, j2n, j3n


def _merge_pop(a, b):
    (ad, aj), (bd, bj) = a, b
    c = bd[0] < ad[0]
    od = jnp.where(c, bd[0], ad[0])
    oj = jnp.where(c, bj[0], aj[0])
    an = ([jnp.where(c, x, y) for x, y in zip(ad[:-1], ad[1:])],
          [jnp.where(c, x, y) for x, y in zip(aj[:-1], aj[1:])])
    bn = ([jnp.where(c, y, x) for x, y in zip(bd[:-1], bd[1:])],
          [jnp.where(c, y, x) for x, y in zip(bj[:-1], bj[1:])])
    return od, oj, an, bn


def _sc_knn_body(posx_h, posy_h, posz_h, qx_h, qy_h, qz_h, qb_h,
                 sp_h, ep_h, sh_h, glo_h, ghi_h, x_h, out_h,
                 posx, posy, posz, qx, qy, qz, qb, sp_r, ep_r, sh_r,
                 glo, ghi, wn1, wn2, wn3, i1, i2, i3,
                 rows1, rows2, rows3, obuf, sem):
    wid = lax.axis_index("s") * NC + lax.axis_index("c")
    base = wid * QPW

    pltpu.sync_copy(posx_h, posx)
    pltpu.sync_copy(posy_h, posy)
    pltpu.sync_copy(posz_h, posz)
    pltpu.sync_copy(qx_h.at[pl.ds(base, QPW)], qx)
    pltpu.sync_copy(qy_h.at[pl.ds(base, QPW)], qy)
    pltpu.sync_copy(qz_h.at[pl.ds(base, QPW)], qz)
    pltpu.sync_copy(qb_h.at[pl.ds(base, QPW)], qb)
    pltpu.sync_copy(sp_h, sp_r)
    pltpu.sync_copy(ep_h, ep_r)
    pltpu.sync_copy(sh_h, sh_r)
    pltpu.sync_copy(glo_h.at[pl.ds(wid * NG, NG)], glo.at[pl.ds(0, NG)])
    pltpu.sync_copy(ghi_h.at[pl.ds(wid * NG, NG)], ghi.at[pl.ds(0, NG)])

    spv = sp_r[pl.ds(0, L)]
    epv = ep_r[pl.ds(0, L)]
    shv = sh_r[pl.ds(0, L)]

    finf = jnp.full((L,), INF, jnp.float32)
    zi = jnp.zeros((L,), jnp.int32)

    @pl.loop(0, NG)
    def _group(g):
        gb = g * L
        qxg = qx[pl.ds(gb, L)]
        qyg = qy[pl.ds(gb, L)]
        qzg = qz[pl.ds(gb, L)]
        bg = qb[pl.ds(gb, L)]
        lob = _ext(glo[pl.ds(g, L)], 0)
        hib = _ext(ghi[pl.ds(g, L)], 0)
        uniform = _ext(bg, 0) == _ext(bg, L - 1)

        def scan(masked):
            if masked:
                los = _perm(spv, bg)
                his = _perm(epv, bg)

            init = (finf, finf, finf, zi, zi, zi,
                    finf, finf, finf, zi, zi, zi)

            @pl.loop(lob, hib, init_carry=init)
            def _blk(jb, carry):
                (d1a, d2a, d3a, j1a, j2a, j3a,
                 d1b, d2b, d3b, j1b, j2b, j3b) = carry
                jb16 = jb * L
                px16 = posx[pl.ds(jb16, L)]
                py16 = posy[pl.ds(jb16, L)]
                pz16 = posz[pl.ds(jb16, L)]
                jvb = jnp.full((L,), jb16, jnp.int32)

                def dist(t):
                    tv = jnp.full((L,), t, jnp.int32)
                    px = _perm(px16, tv)
                    py = _perm(py16, tv)
                    pz = _perm(pz16, tv)
                    jv = jvb + t
                    dx = qxg - px
                    dy = qyg - py
                    dz = qzg - pz
                    d = dx * dx + dy * dy + dz * dz
                    if masked:
                        d = jnp.where((jv >= los) & (jv < his), d, INF)
                    return d, jv

                for t in range(0, L, 2):
                    da, ja = dist(t)
                    db, jb2 = dist(t + 1)
                    d1a, d2a, d3a, j1a, j2a, j3a = _top3_update(
                        da, ja, d1a, d2a, d3a, j1a, j2a, j3a)
                    d1b, d2b, d3b, j1b, j2b, j3b = _top3_update(
                        db, jb2, d1b, d2b, d3b, j1b, j2b, j3b)
                return (d1a, d2a, d3a, j1a, j2a, j3a,
                        d1b, d2b, d3b, j1b, j2b, j3b)

            (d1a, d2a, d3a, j1a, j2a, j3a,
             d1b, d2b, d3b, j1b, j2b, j3b) = _blk

            a = ([d1a, d2a, d3a], [j1a, j2a, j3a])
            b = ([d1b, d2b, d3b], [j1b, j2b, j3b])
            o1d, o1j, a2_, b2_ = _merge_pop(a, b)
            o2d, o2j, a3_, b3_ = _merge_pop(a2_, b2_)
            c = b3_[0][0] < a3_[0][0]
            o3d = jnp.where(c, b3_[0][0], a3_[0][0])
            o3j = jnp.where(c, b3_[1][0], a3_[1][0])

            jadj = _perm(shv, bg)
            o1jc = jnp.clip(o1j - jadj, 0, NX - 1)
            o2jc = jnp.clip(o2j - jadj, 0, NX - 1)
            o3jc = jnp.clip(o3j - jadj, 0, NX - 1)

            w1 = 1.0 / jnp.maximum(o1d, 1e-16)
            w2 = 1.0 / jnp.maximum(o2d, 1e-16)
            w3 = 1.0 / jnp.maximum(o3d, 1e-16)
            r = 1.0 / (w1 + w2 + w3)
            wn1[pl.ds(gb, L)] = w1 * r
            wn2[pl.ds(gb, L)] = w2 * r
            wn3[pl.ds(gb, L)] = w3 * r
            i1[pl.ds(gb, L)] = o1jc
            i2[pl.ds(gb, L)] = o2jc
            i3[pl.ds(gb, L)] = o3jc

        lax.cond(uniform, lambda: scan(False), lambda: scan(True))

    zsplat = jnp.zeros((L,), jnp.int32)
    for c in range(NCHUNK):
        cb = c * CHUNK
        cp1 = pltpu.async_copy(x_h.at[i1.at[pl.ds(cb, CHUNK)]], rows1, sem)
        cp2 = pltpu.async_copy(x_h.at[i2.at[pl.ds(cb, CHUNK)]], rows2, sem)
        cp3 = pltpu.async_copy(x_h.at[i3.at[pl.ds(cb, CHUNK)]], rows3, sem)
        cp1.wait()
        cp2.wait()
        cp3.wait()

        @pl.loop(0, CHUNK)
        def _combine(q):
            a1 = _perm(wn1[pl.ds(cb + q, L)], zsplat)
            a2 = _perm(wn2[pl.ds(cb + q, L)], zsplat)
            a3 = _perm(wn3[pl.ds(cb + q, L)], zsplat)
            for k in range(D // L):
                kk = pl.ds(k * L, L)
                obuf[q, kk] = (a1 * rows1[q, kk] + a2 * rows2[q, kk]
                               + a3 * rows3[q, kk])

        pltpu.sync_copy(obuf, out_h.at[pl.ds(base + cb, CHUNK)])


def _sc_knn(posx, posy, posz, qx, qy, qz, qb, sp, ep, sh, glo, ghi, x):
    mesh = plsc.VectorSubcoreMesh(core_axis_name="c", subcore_axis_name="s",
                                  num_cores=NC, num_subcores=NS)
    f = pl.kernel(
        _sc_knn_body,
        out_type=jax.ShapeDtypeStruct((NQ, D), jnp.float32),
        mesh=mesh,
        scratch_types=[
            pltpu.VMEM((NXP,), jnp.float32),
            pltpu.VMEM((NXP,), jnp.float32),
            pltpu.VMEM((NXP,), jnp.float32),
            pltpu.VMEM((QPW,), jnp.float32),
            pltpu.VMEM((QPW,), jnp.float32),
            pltpu.VMEM((QPW,), jnp.float32),
            pltpu.VMEM((QPW,), jnp.int32),
            pltpu.VMEM((L,), jnp.int32),
            pltpu.VMEM((L,), jnp.int32),
            pltpu.VMEM((L,), jnp.int32),
            pltpu.VMEM((NG + L,), jnp.int32),
            pltpu.VMEM((NG + L,), jnp.int32),
            pltpu.VMEM((QPW + L,), jnp.float32),
            pltpu.VMEM((QPW + L,), jnp.float32),
            pltpu.VMEM((QPW + L,), jnp.float32),
            pltpu.VMEM((QPW,), jnp.int32),
            pltpu.VMEM((QPW,), jnp.int32),
            pltpu.VMEM((QPW,), jnp.int32),
            pltpu.VMEM((CHUNK, D), jnp.float32),
            pltpu.VMEM((CHUNK, D), jnp.float32),
            pltpu.VMEM((CHUNK, D), jnp.float32),
            pltpu.VMEM((CHUNK, D), jnp.float32),
            pltpu.SemaphoreType.DMA,
        ],
    )
    return f(posx, posy, posz, qx, qy, qz, qb, sp, ep, sh, glo, ghi, x)


def _tc_mlp_body(xi_ref, xs_ref, pe_ref, w1a_ref, w1b_ref, b1_ref,
                 w2_ref, b2_ref, wp_ref, bp_ref, out_ref):
    i = pl.program_id(0)
    pe_row = pe_ref[pl.ds(i, 1), :]
    pr = jnp.dot(pe_row, wp_ref[...],
                 preferred_element_type=jnp.float32) + bp_ref[...]
    pr = jnp.maximum(pr, 0.0)
    h = jnp.dot(xi_ref[...], w1a_ref[...], preferred_element_type=jnp.float32)
    h = h + jnp.dot(xs_ref[...], w1b_ref[...],
                    preferred_element_type=jnp.float32)
    h = jnp.maximum(h + b1_ref[...], 0.0)
    h = jnp.dot(h, w2_ref[...], preferred_element_type=jnp.float32)
    out_ref[...] = (h + b2_ref[...]) * pr


def _tc_mlp(xi, xs, pe, w1a, w1b, b1, w2, b2, wp, bp):
    nb = 8
    rb = NQ // nb
    return pl.pallas_call(
        _tc_mlp_body,
        grid=(nb,),
        in_specs=[
            pl.BlockSpec((rb, 128), lambda i: (i, 0)),
            pl.BlockSpec((rb, 64), lambda i: (i, 0)),
            pl.BlockSpec((8, 512), lambda i: (0, 0)),
            pl.BlockSpec((128, 128), lambda i: (0, 0)),
            pl.BlockSpec((64, 128), lambda i: (0, 0)),
            pl.BlockSpec((1, 128), lambda i: (0, 0)),
            pl.BlockSpec((128, 128), lambda i: (0, 0)),
            pl.BlockSpec((1, 128), lambda i: (0, 0)),
            pl.BlockSpec((512, 128), lambda i: (0, 0)),
            pl.BlockSpec((1, 128), lambda i: (0, 0)),
        ],
        out_specs=pl.BlockSpec((rb, 128), lambda i: (i, 0)),
        out_shape=jax.ShapeDtypeStruct((NQ, 128), jnp.float32),
    )(xi, xs, pe, w1a, w1b, b1, w2, b2, wp, bp)


def kernel(par_embedding, x, pos, batch, x_skip, pos_skip, batch_skip,
           W1, b1, W2, b2, Wp, bp):
    batch = batch.astype(jnp.int32)
    qb = batch_skip.astype(jnp.int32)
    qx = pos_skip[:, 0] + 0.0
    qy = pos_skip[:, 1] + 0.0
    qz = pos_skip[:, 2] + 0.0
    ar = jnp.arange(8, dtype=jnp.int32)
    ss = jnp.searchsorted(batch, ar, side="left").astype(jnp.int32)
    se = jnp.searchsorted(batch, ar, side="right").astype(jnp.int32)
    cnt = se - ss
    cntp = ((cnt + L - 1) // L) * L
    off = jnp.concatenate([jnp.zeros((1,), jnp.int32),
                           jnp.cumsum(cntp)[:-1].astype(jnp.int32)])
    shift = off - ss
    dst = jnp.arange(NX, dtype=jnp.int32) + shift[batch]
    sent = jnp.full((NXP,), SENT, jnp.float32)
    posx = sent.at[dst].set(pos[:, 0])
    posy = sent.at[dst].set(pos[:, 1])
    posz = sent.at[dst].set(pos[:, 2])
    sp16 = jnp.pad(off, (0, L - 8))
    ep16 = jnp.pad(off + cnt, (0, L - 8))
    sh16 = jnp.pad(shift, (0, L - 8))
    bs = qb.reshape(NQ // L, L)
    glo = jnp.pad(off[bs[:, 0]] // L, (0, L))
    ghi = jnp.pad((off + cntp)[bs[:, L - 1]] // L, (0, L))

    xi = _sc_knn(posx, posy, posz, qx, qy, qz, qb,
                 sp16, ep16, sh16, glo, ghi, x)

    pe = par_embedding.reshape(8, 512)
    w1a = W1[:128]
    w1b = W1[128:]
    out = _tc_mlp(xi, x_skip, pe, w1a, w1b, b1.reshape(1, 128),
                  W2, b2.reshape(1, 128), Wp, bp.reshape(1, 128))
    return out, pos_skip, batch_skip
